# baseline (speedup 1.0000x reference)
import functools

import jax
import jax.numpy as jnp
from jax import lax
from jax.experimental import pallas as pl
from jax.experimental.pallas import tpu as pltpu
from jax.experimental.pallas import tpu_sc as plsc

_N = 10000
_D = 128
_DH = _D // 2
_E = 320000
_NC = 2
_NS = 16
_B = 80
_CHUNKS = _E // (_NS * _B)
_RING = 5
_RSTRIDE = 624
_WR = 640
_CW = 16

_mesh = plsc.VectorSubcoreMesh(
    core_axis_name="c", subcore_axis_name="s", num_cores=_NC, num_subcores=_NS
)


@functools.partial(
    pl.kernel,
    out_type=(
        jax.ShapeDtypeStruct((_NC, _N, _DH), jnp.float32),
        jax.ShapeDtypeStruct((_NC, _N, _CW), jnp.float32),
    ),
    mesh=_mesh,
    compiler_params=pltpu.CompilerParams(use_tc_tiling_on_sc=False),
    scratch_types=[
        pltpu.VMEM((_CHUNKS, _B), jnp.int32),
        pltpu.VMEM((_CHUNKS, _B), jnp.int32),
        [pltpu.VMEM((_B, _DH), jnp.float32) for _ in range(_RING)],
        pltpu.VMEM((_B, _CW), jnp.float32),
        pltpu.VMEM((_B, _DH), jnp.float32),
        pltpu.VMEM((_B, _CW), jnp.float32),
        pltpu.VMEM_SHARED((_N, _DH), jnp.float32),
        pltpu.VMEM_SHARED((_N, _CW), jnp.float32),
        [pltpu.SemaphoreType.DMA for _ in range(_RING)],
        [pltpu.SemaphoreType.DMA for _ in range(_RING)],
        pltpu.SemaphoreType.DMA,
    ],
)
def _sc_aggregate(nodes_halves, senders3d, receivers3d, out_sums, out_cnts,
                  sidx, ridx, rows, ones_v, zrow, zcnt,
                  acc, cac, gsem, ssem, csem):
    cid = lax.axis_index("c")
    sid = lax.axis_index("s")

    zero16 = jnp.zeros((_CW,), jnp.float32)
    one16 = jnp.ones((_CW,), jnp.float32)

    def init_body(r, carry):
        ones_v[r, :] = one16
        zcnt[r, :] = zero16
        for k in range(_DH // 16):
            zrow[r, pl.ds(k * 16, 16)] = zero16
        return carry

    lax.fori_loop(0, _B, init_body, 0)

    base = sid * _RSTRIDE
    pltpu.async_copy(senders3d.at[sid], sidx, gsem[0])
    pltpu.async_copy(receivers3d.at[sid], ridx, gsem[1])
    for i in range(_WR // _B):
        st = base + i * _B
        pltpu.async_copy(zrow, acc.at[pl.ds(st, _B)], ssem[0])
        pltpu.async_copy(zcnt, cac.at[pl.ds(st, _B)], ssem[1])
    pltpu.make_async_copy(senders3d.at[sid], sidx, gsem[0]).wait()
    pltpu.make_async_copy(receivers3d.at[sid], ridx, gsem[1]).wait()
    for i in range(_WR // _B):
        pltpu.make_async_copy(zrow, acc.at[pl.ds(base, _B)], ssem[0]).wait()
        pltpu.make_async_copy(zcnt, cac.at[pl.ds(base, _B)], ssem[1]).wait()
    plsc.subcore_barrier()

    src = nodes_halves.at[cid]

    def start_gather(j, b):
        pltpu.async_copy(src.at[sidx.at[j]], rows[b], gsem[b])

    def wait_gather(b):
        pltpu.make_async_copy(src.at[sidx.at[0]], rows[b], gsem[b]).wait()

    def start_scat(j, b):
        pltpu.async_copy(rows[b], acc.at[ridx.at[j]], ssem[b], add=True)

    def wait_scat(b):
        pltpu.make_async_copy(rows[b], acc.at[ridx.at[0]], ssem[b]).wait()

    def count_scat(j):
        @pl.when(lax.rem(j, 2) == cid)
        def _():
            pltpu.async_copy(ones_v, cac.at[ridx.at[j]], csem, add=True)
            pltpu.make_async_copy(ones_v, cac.at[ridx.at[0]], csem).wait()

    for b in range(_RING):
        start_gather(b, b)

    def body(i, carry):
        j0 = _RING * i
        for b in range(_RING):
            wait_gather(b)
            start_scat(j0 + b, b)
            count_scat(j0 + b)
        for b in range(_RING):
            wait_scat(b)
            start_gather(j0 + b + _RING, b)
        return carry

    lax.fori_loop(0, _CHUNKS // _RING - 1, body, 0)
    for b in range(_RING):
        wait_gather(b)
        start_scat(_CHUNKS - _RING + b, b)
        count_scat(_CHUNKS - _RING + b)
    for b in range(_RING):
        wait_scat(b)

    plsc.subcore_barrier()

    pltpu.async_copy(acc.at[pl.ds(base, _WR)],
                     out_sums.at[cid, pl.ds(base, _WR)], gsem[0])
    pltpu.async_copy(cac.at[pl.ds(base, _WR)],
                     out_cnts.at[cid, pl.ds(base, _WR)], gsem[1])
    pltpu.make_async_copy(acc.at[pl.ds(base, _WR)],
                          out_sums.at[cid, pl.ds(base, _WR)], gsem[0]).wait()
    pltpu.make_async_copy(cac.at[pl.ds(base, _WR)],
                          out_cnts.at[cid, pl.ds(base, _WR)], gsem[1]).wait()


_MB = 1000


def _tc_body(res_ref, x_ref, s_ref, c_ref, w_ref, b_ref, o_ref):
    x = x_ref[...]
    c = c_ref[0] + c_ref[1]
    cnt = jnp.maximum(c[:, 0:1], 1.0)
    mean_lo = s_ref[0] / cnt
    mean_hi = s_ref[1] / cnt
    w = w_ref[...]
    acc = jnp.dot(x, w[0:_D], preferred_element_type=jnp.float32,
                  precision=lax.Precision.HIGHEST)
    acc = acc + jnp.dot(mean_lo, w[_D:_D + _DH],
                        preferred_element_type=jnp.float32,
                        precision=lax.Precision.HIGHEST)
    acc = acc + jnp.dot(mean_hi, w[_D + _DH:2 * _D],
                        preferred_element_type=jnp.float32,
                        precision=lax.Precision.HIGHEST)
    o_ref[...] = acc + b_ref[...] + res_ref[0, 0]


def _tc_finish(res, nodes, sums, cnts, W, b2d):
    return pl.pallas_call(
        _tc_body,
        grid=(_N // _MB,),
        in_specs=[
            pl.BlockSpec(memory_space=pltpu.SMEM),
            pl.BlockSpec((_MB, _D), lambda i: (i, 0)),
            pl.BlockSpec((_NC, _MB, _DH), lambda i: (0, i, 0)),
            pl.BlockSpec((_NC, _MB, _CW), lambda i: (0, i, 0)),
            pl.BlockSpec((2 * _D, _D), lambda i: (0, 0)),
            pl.BlockSpec((1, _D), lambda i: (0, 0)),
        ],
        out_specs=pl.BlockSpec((_MB, _D), lambda i: (i, 0)),
        out_shape=jax.ShapeDtypeStruct((_N, _D), jnp.float32),
    )(res, nodes, sums, cnts, W, b2d)


def kernel(nodes, senders, receivers, n_node, W, b):
    nodes_halves = jnp.stack([nodes[:, :_DH], nodes[:, _DH:]])
    senders3d = senders.reshape(_NS, _CHUNKS, _B)
    receivers3d = receivers.reshape(_NS, _CHUNKS, _B)
    sums, cnts = _sc_aggregate(nodes_halves, senders3d, receivers3d)
    res = (jnp.asarray(n_node, jnp.float32) - jnp.float32(_N)).reshape(1, 1)
    return _tc_finish(res, nodes, sums, cnts, W, b.reshape(1, _D))


# speedup vs baseline: 1.1665x; 1.1665x over previous
import functools

import jax
import jax.numpy as jnp
from jax import lax
from jax.experimental import pallas as pl
from jax.experimental.pallas import tpu as pltpu
from jax.experimental.pallas import tpu_sc as plsc

_N = 10000
_D = 128
_DH = _D // 2
_E = 320000
_NC = 2
_NS = 16
_B = 80
_CHUNKS = _E // (_NS * _B)
_RING = 5
_RSTRIDE = 624
_WR = 640
_CW = 16

_mesh = plsc.VectorSubcoreMesh(
    core_axis_name="c", subcore_axis_name="s", num_cores=_NC, num_subcores=_NS
)


@functools.partial(
    pl.kernel,
    out_type=(
        jax.ShapeDtypeStruct((_NC, _N, _DH), jnp.float32),
        jax.ShapeDtypeStruct((_NC, _N, _CW), jnp.float32),
    ),
    mesh=_mesh,
    compiler_params=pltpu.CompilerParams(use_tc_tiling_on_sc=False),
    scratch_types=[
        pltpu.VMEM((_CHUNKS, _B), jnp.int32),
        pltpu.VMEM((_CHUNKS, _B), jnp.int32),
        [pltpu.VMEM((_B, _DH), jnp.float32) for _ in range(_RING)],
        pltpu.VMEM((_B, _CW), jnp.float32),
        pltpu.VMEM((_B, _DH), jnp.float32),
        pltpu.VMEM((_B, _CW), jnp.float32),
        pltpu.VMEM_SHARED((_N, _DH), jnp.float32),
        pltpu.VMEM_SHARED((_N, _CW), jnp.float32),
        [pltpu.SemaphoreType.DMA for _ in range(_RING)],
        [pltpu.SemaphoreType.DMA for _ in range(_RING)],
    ],
)
def _sc_aggregate(nodes_halves, senders3d, receivers3d, out_sums, out_cnts,
                  sidx, ridx, rows, ones_v, zrow, zcnt,
                  acc, cac, gsem, ssem):
    cid = lax.axis_index("c")
    sid = lax.axis_index("s")

    zero16 = jnp.zeros((_CW,), jnp.float32)
    one16 = jnp.ones((_CW,), jnp.float32)

    def init_body(r, carry):
        ones_v[r, :] = one16
        zcnt[r, :] = zero16
        for k in range(_DH // 16):
            zrow[r, pl.ds(k * 16, 16)] = zero16
        return carry

    lax.fori_loop(0, _B, init_body, 0)

    base = sid * _RSTRIDE
    pltpu.async_copy(senders3d.at[sid], sidx, gsem[0])
    pltpu.async_copy(receivers3d.at[sid], ridx, gsem[1])
    for i in range(_WR // _B):
        st = base + i * _B
        pltpu.async_copy(zrow, acc.at[pl.ds(st, _B)], ssem[0])
        pltpu.async_copy(zcnt, cac.at[pl.ds(st, _B)], ssem[1])
    pltpu.make_async_copy(senders3d.at[sid], sidx, gsem[0]).wait()
    pltpu.make_async_copy(receivers3d.at[sid], ridx, gsem[1]).wait()
    for i in range(_WR // _B):
        pltpu.make_async_copy(zrow, acc.at[pl.ds(base, _B)], ssem[0]).wait()
        pltpu.make_async_copy(zcnt, cac.at[pl.ds(base, _B)], ssem[1]).wait()
    plsc.subcore_barrier()

    src = nodes_halves.at[cid]

    def start_gather(j, b):
        pltpu.async_copy(src.at[sidx.at[j]], rows[b], gsem[b])

    def wait_gather(b):
        pltpu.make_async_copy(src.at[sidx.at[0]], rows[b], gsem[b]).wait()

    def start_scat(j, b):
        pltpu.async_copy(rows[b], acc.at[ridx.at[j]], ssem[b], add=True)

        @pl.when(lax.rem(j, 2) == cid)
        def _():
            pltpu.async_copy(ones_v, cac.at[ridx.at[j]], ssem[b], add=True)

    def wait_scat(j, b):
        pltpu.make_async_copy(rows[b], acc.at[ridx.at[0]], ssem[b]).wait()

        @pl.when(lax.rem(j, 2) == cid)
        def _():
            pltpu.make_async_copy(ones_v, cac.at[ridx.at[0]], ssem[b]).wait()

    for b in range(_RING):
        start_gather(b, b)

    def body(i, carry):
        j0 = _RING * i
        for b in range(_RING):
            wait_gather(b)
            start_scat(j0 + b, b)
        for b in range(_RING):
            wait_scat(j0 + b, b)
            start_gather(j0 + b + _RING, b)
        return carry

    lax.fori_loop(0, _CHUNKS // _RING - 1, body, 0)
    for b in range(_RING):
        wait_gather(b)
        start_scat(_CHUNKS - _RING + b, b)
    for b in range(_RING):
        wait_scat(_CHUNKS - _RING + b, b)

    plsc.subcore_barrier()

    pltpu.async_copy(acc.at[pl.ds(base, _WR)],
                     out_sums.at[cid, pl.ds(base, _WR)], gsem[0])
    pltpu.async_copy(cac.at[pl.ds(base, _WR)],
                     out_cnts.at[cid, pl.ds(base, _WR)], gsem[1])
    pltpu.make_async_copy(acc.at[pl.ds(base, _WR)],
                          out_sums.at[cid, pl.ds(base, _WR)], gsem[0]).wait()
    pltpu.make_async_copy(cac.at[pl.ds(base, _WR)],
                          out_cnts.at[cid, pl.ds(base, _WR)], gsem[1]).wait()


_MB = 1000


def _tc_body(res_ref, x_ref, s_ref, c_ref, w_ref, b_ref, o_ref):
    x = x_ref[...]
    c = c_ref[0] + c_ref[1]
    cnt = jnp.maximum(c[:, 0:1], 1.0)
    mean_lo = s_ref[0] / cnt
    mean_hi = s_ref[1] / cnt
    w = w_ref[...]
    acc = jnp.dot(x, w[0:_D], preferred_element_type=jnp.float32,
                  precision=lax.Precision.HIGHEST)
    acc = acc + jnp.dot(mean_lo, w[_D:_D + _DH],
                        preferred_element_type=jnp.float32,
                        precision=lax.Precision.HIGHEST)
    acc = acc + jnp.dot(mean_hi, w[_D + _DH:2 * _D],
                        preferred_element_type=jnp.float32,
                        precision=lax.Precision.HIGHEST)
    o_ref[...] = acc + b_ref[...] + res_ref[0, 0]


def _tc_finish(res, nodes, sums, cnts, W, b2d):
    return pl.pallas_call(
        _tc_body,
        grid=(_N // _MB,),
        in_specs=[
            pl.BlockSpec(memory_space=pltpu.SMEM),
            pl.BlockSpec((_MB, _D), lambda i: (i, 0)),
            pl.BlockSpec((_NC, _MB, _DH), lambda i: (0, i, 0)),
            pl.BlockSpec((_NC, _MB, _CW), lambda i: (0, i, 0)),
            pl.BlockSpec((2 * _D, _D), lambda i: (0, 0)),
            pl.BlockSpec((1, _D), lambda i: (0, 0)),
        ],
        out_specs=pl.BlockSpec((_MB, _D), lambda i: (i, 0)),
        out_shape=jax.ShapeDtypeStruct((_N, _D), jnp.float32),
    )(res, nodes, sums, cnts, W, b2d)


def kernel(nodes, senders, receivers, n_node, W, b):
    nodes_halves = jnp.stack([nodes[:, :_DH], nodes[:, _DH:]])
    senders3d = senders.reshape(_NS, _CHUNKS, _B)
    receivers3d = receivers.reshape(_NS, _CHUNKS, _B)
    sums, cnts = _sc_aggregate(nodes_halves, senders3d, receivers3d)
    res = (jnp.asarray(n_node, jnp.float32) - jnp.float32(_N)).reshape(1, 1)
    return _tc_finish(res, nodes, sums, cnts, W, b.reshape(1, _D))


# speedup vs baseline: 1.2595x; 1.0797x over previous
import functools

import jax
import jax.numpy as jnp
from jax import lax
from jax.experimental import pallas as pl
from jax.experimental.pallas import tpu as pltpu
from jax.experimental.pallas import tpu_sc as plsc

_N = 10000
_D = 128
_DH = _D // 2
_E = 320000
_NC = 2
_NS = 16
_B = 80
_CHUNKS = _E // (_NS * _B)
_RING = 5
_RSTRIDE = 624
_WR = 640
_CW = 16

_mesh = plsc.VectorSubcoreMesh(
    core_axis_name="c", subcore_axis_name="s", num_cores=_NC, num_subcores=_NS
)


@functools.partial(
    pl.kernel,
    out_type=(
        jax.ShapeDtypeStruct((_NC, _N, _DH), jnp.float32),
        jax.ShapeDtypeStruct((_NC, _N, _CW), jnp.float32),
    ),
    mesh=_mesh,
    compiler_params=pltpu.CompilerParams(use_tc_tiling_on_sc=False),
    scratch_types=[
        pltpu.VMEM((_CHUNKS, _B), jnp.int32),
        pltpu.VMEM((_CHUNKS, _B), jnp.int32),
        [pltpu.VMEM((_B, _DH), jnp.float32) for _ in range(_RING)],
        pltpu.VMEM((_B, _CW), jnp.float32),
        pltpu.VMEM((_B, _DH), jnp.float32),
        pltpu.VMEM((_B, _CW), jnp.float32),
        pltpu.VMEM_SHARED((_N, _DH), jnp.float32),
        pltpu.VMEM_SHARED((_N, _CW), jnp.float32),
        [pltpu.SemaphoreType.DMA for _ in range(_RING)],
        [pltpu.SemaphoreType.DMA for _ in range(_RING)],
    ],
)
def _sc_aggregate(nodes_halves, senders3d, receivers3d, out_sums, out_cnts,
                  sidx, ridx, rows, ones_v, zrow, zcnt,
                  acc, cac, gsem, ssem):
    cid = lax.axis_index("c")
    sid = lax.axis_index("s")

    zero16 = jnp.zeros((_CW,), jnp.float32)
    one16 = jnp.ones((_CW,), jnp.float32)

    def init_body(r, carry):
        ones_v[r, :] = one16
        zcnt[r, :] = zero16
        for k in range(_DH // 16):
            zrow[r, pl.ds(k * 16, 16)] = zero16
        return carry

    lax.fori_loop(0, _B, init_body, 0)

    base = sid * _RSTRIDE
    pltpu.async_copy(senders3d.at[sid], sidx, gsem[0])
    pltpu.async_copy(receivers3d.at[sid], ridx, gsem[1])
    for i in range(_WR // _B):
        st = base + i * _B
        pltpu.async_copy(zrow, acc.at[pl.ds(st, _B)], ssem[0])
        pltpu.async_copy(zcnt, cac.at[pl.ds(st, _B)], ssem[1])
    pltpu.make_async_copy(senders3d.at[sid], sidx, gsem[0]).wait()
    pltpu.make_async_copy(receivers3d.at[sid], ridx, gsem[1]).wait()
    for i in range(_WR // _B):
        pltpu.make_async_copy(zrow, acc.at[pl.ds(base, _B)], ssem[0]).wait()
        pltpu.make_async_copy(zcnt, cac.at[pl.ds(base, _B)], ssem[1]).wait()
    plsc.subcore_barrier()

    src = nodes_halves.at[cid]

    def start_gather(j, b):
        pltpu.async_copy(src.at[sidx.at[j]], rows[b], gsem[b])

    def wait_gather(b):
        pltpu.make_async_copy(src.at[sidx.at[0]], rows[b], gsem[b]).wait()

    def start_scat(j, b):
        pltpu.async_copy(rows[b], acc.at[ridx.at[j]], ssem[b], add=True)

        @pl.when(lax.rem(j, 2) == cid)
        def _():
            pltpu.async_copy(ones_v, cac.at[ridx.at[j]], ssem[b], add=True)

    def wait_scat(j, b):
        pltpu.make_async_copy(rows[b], acc.at[ridx.at[0]], ssem[b]).wait()

        @pl.when(lax.rem(j, 2) == cid)
        def _():
            pltpu.make_async_copy(ones_v, cac.at[ridx.at[0]], ssem[b]).wait()

    for b in range(_RING):
        start_gather(b, b)

    def body(i, carry):
        j0 = _RING * i
        for b in range(_RING):
            wait_gather(b)
            start_scat(j0 + b, b)
        for b in range(_RING):
            wait_scat(j0 + b, b)
            start_gather(j0 + b + _RING, b)
        return carry

    lax.fori_loop(0, _CHUNKS // _RING - 1, body, 0)
    for b in range(_RING):
        wait_gather(b)
        start_scat(_CHUNKS - _RING + b, b)
    for b in range(_RING):
        wait_scat(_CHUNKS - _RING + b, b)

    plsc.subcore_barrier()

    pltpu.async_copy(acc.at[pl.ds(base, _WR)],
                     out_sums.at[cid, pl.ds(base, _WR)], gsem[0])
    pltpu.async_copy(cac.at[pl.ds(base, _WR)],
                     out_cnts.at[cid, pl.ds(base, _WR)], gsem[1])
    pltpu.make_async_copy(acc.at[pl.ds(base, _WR)],
                          out_sums.at[cid, pl.ds(base, _WR)], gsem[0]).wait()
    pltpu.make_async_copy(cac.at[pl.ds(base, _WR)],
                          out_cnts.at[cid, pl.ds(base, _WR)], gsem[1]).wait()


_MB = 1000


def _tc_body(res_ref, x_ref, s_ref, c_ref, w_ref, b_ref, o_ref):
    x = x_ref[...]
    c = c_ref[0] + c_ref[1]
    cnt = jnp.maximum(c[:, 0:1], 1.0)
    mean_lo = s_ref[0] / cnt
    mean_hi = s_ref[1] / cnt
    w = w_ref[...]
    acc = jnp.dot(x, w[0:_D], preferred_element_type=jnp.float32)
    acc = acc + jnp.dot(mean_lo, w[_D:_D + _DH],
                        preferred_element_type=jnp.float32)
    acc = acc + jnp.dot(mean_hi, w[_D + _DH:2 * _D],
                        preferred_element_type=jnp.float32)
    o_ref[...] = acc + b_ref[...] + res_ref[0, 0]


def _tc_finish(res, nodes, sums, cnts, W, b2d):
    return pl.pallas_call(
        _tc_body,
        grid=(_N // _MB,),
        in_specs=[
            pl.BlockSpec(memory_space=pltpu.SMEM),
            pl.BlockSpec((_MB, _D), lambda i: (i, 0)),
            pl.BlockSpec((_NC, _MB, _DH), lambda i: (0, i, 0)),
            pl.BlockSpec((_NC, _MB, _CW), lambda i: (0, i, 0)),
            pl.BlockSpec((2 * _D, _D), lambda i: (0, 0)),
            pl.BlockSpec((1, _D), lambda i: (0, 0)),
        ],
        out_specs=pl.BlockSpec((_MB, _D), lambda i: (i, 0)),
        out_shape=jax.ShapeDtypeStruct((_N, _D), jnp.float32),
    )(res, nodes, sums, cnts, W, b2d)


def kernel(nodes, senders, receivers, n_node, W, b):
    nodes_halves = jnp.stack([nodes[:, :_DH], nodes[:, _DH:]])
    senders3d = senders.reshape(_NS, _CHUNKS, _B)
    receivers3d = receivers.reshape(_NS, _CHUNKS, _B)
    sums, cnts = _sc_aggregate(nodes_halves, senders3d, receivers3d)
    res = (jnp.asarray(n_node, jnp.float32) - jnp.float32(_N)).reshape(1, 1)
    return _tc_finish(res, nodes, sums, cnts, W, b.reshape(1, _D))


# speedup vs baseline: 1.2813x; 1.0173x over previous
import functools

import jax
import jax.numpy as jnp
from jax import lax
from jax.experimental import pallas as pl
from jax.experimental.pallas import tpu as pltpu
from jax.experimental.pallas import tpu_sc as plsc

_N = 10000
_D = 128
_DH = _D // 2
_E = 320000
_NC = 2
_NS = 16
_B = 80
_CHUNKS = _E // (_NS * _B)
_RING = 5
_RSTRIDE = 624
_WR = 640
_CW = 16

_mesh = plsc.VectorSubcoreMesh(
    core_axis_name="c", subcore_axis_name="s", num_cores=_NC, num_subcores=_NS
)


@functools.partial(
    pl.kernel,
    out_type=(
        jax.ShapeDtypeStruct((_NC, _N, _DH), jnp.float32),
        jax.ShapeDtypeStruct((_NC, _N, _CW), jnp.float32),
    ),
    mesh=_mesh,
    compiler_params=pltpu.CompilerParams(use_tc_tiling_on_sc=False),
    scratch_types=[
        pltpu.VMEM((_CHUNKS, _B), jnp.int32),
        pltpu.VMEM((_CHUNKS, _B), jnp.int32),
        [pltpu.VMEM((_B, _DH), jnp.float32) for _ in range(_RING)],
        pltpu.VMEM((_B, _CW), jnp.float32),
        pltpu.VMEM((_B, _DH), jnp.float32),
        pltpu.VMEM((_B, _CW), jnp.float32),
        pltpu.VMEM_SHARED((_N, _DH), jnp.float32),
        pltpu.VMEM_SHARED((_N, _CW), jnp.float32),
        [pltpu.SemaphoreType.DMA for _ in range(_RING)],
        [pltpu.SemaphoreType.DMA for _ in range(_RING)],
    ],
)
def _sc_aggregate(nodes_halves, senders3d, receivers3d, out_sums, out_cnts,
                  sidx, ridx, rows, ones_v, zrow, zcnt,
                  acc, cac, gsem, ssem):
    cid = lax.axis_index("c")
    sid = lax.axis_index("s")

    zero16 = jnp.zeros((_CW,), jnp.float32)
    one16 = jnp.ones((_CW,), jnp.float32)

    def init_body(r, carry):
        ones_v[r, :] = one16
        zcnt[r, :] = zero16
        for k in range(_DH // 16):
            zrow[r, pl.ds(k * 16, 16)] = zero16
        return carry

    lax.fori_loop(0, _B, init_body, 0)

    base = sid * _RSTRIDE
    pltpu.async_copy(senders3d.at[sid], sidx, gsem[0])
    pltpu.async_copy(receivers3d.at[sid], ridx, gsem[1])
    for i in range(_WR // _B):
        st = base + i * _B
        pltpu.async_copy(zrow, acc.at[pl.ds(st, _B)], ssem[0])
        pltpu.async_copy(zcnt, cac.at[pl.ds(st, _B)], ssem[1])
    pltpu.make_async_copy(senders3d.at[sid], sidx, gsem[0]).wait()
    pltpu.make_async_copy(receivers3d.at[sid], ridx, gsem[1]).wait()
    for i in range(_WR // _B):
        pltpu.make_async_copy(zrow, acc.at[pl.ds(base, _B)], ssem[0]).wait()
        pltpu.make_async_copy(zcnt, cac.at[pl.ds(base, _B)], ssem[1]).wait()
    plsc.subcore_barrier()

    src = nodes_halves.at[cid]

    def start_gather(j, b):
        pltpu.async_copy(src.at[sidx.at[j]], rows[b], gsem[b])

    def wait_gather(b):
        pltpu.make_async_copy(src.at[sidx.at[0]], rows[b], gsem[b]).wait()

    def start_scat(j, b):
        pltpu.async_copy(rows[b], acc.at[ridx.at[j]], ssem[b], add=True)

        @pl.when(lax.rem(j, 2) == cid)
        def _():
            pltpu.async_copy(ones_v, cac.at[ridx.at[j]], ssem[b], add=True)

    def wait_scat(j, b):
        pltpu.make_async_copy(rows[b], acc.at[ridx.at[0]], ssem[b]).wait()

        @pl.when(lax.rem(j, 2) == cid)
        def _():
            pltpu.make_async_copy(ones_v, cac.at[ridx.at[0]], ssem[b]).wait()

    for b in range(_RING):
        start_gather(b, b)

    def body(i, carry):
        j0 = _RING * i
        for b in range(_RING):
            wait_gather(b)
            start_scat(j0 + b, b)
        for b in range(_RING):
            wait_scat(j0 + b, b)
            start_gather(j0 + b + _RING, b)
        return carry

    lax.fori_loop(0, _CHUNKS // _RING - 1, body, 0)
    for b in range(_RING):
        wait_gather(b)
        start_scat(_CHUNKS - _RING + b, b)
    for b in range(_RING):
        wait_scat(_CHUNKS - _RING + b, b)

    plsc.subcore_barrier()

    pltpu.async_copy(acc.at[pl.ds(base, _WR)],
                     out_sums.at[cid, pl.ds(base, _WR)], gsem[0])
    pltpu.async_copy(cac.at[pl.ds(base, _WR)],
                     out_cnts.at[cid, pl.ds(base, _WR)], gsem[1])
    pltpu.make_async_copy(acc.at[pl.ds(base, _WR)],
                          out_sums.at[cid, pl.ds(base, _WR)], gsem[0]).wait()
    pltpu.make_async_copy(cac.at[pl.ds(base, _WR)],
                          out_cnts.at[cid, pl.ds(base, _WR)], gsem[1]).wait()


_MB = 2000


def _tc_body(res_ref, x_ref, s_ref, c_ref, w_ref, b_ref, o_ref):
    x = x_ref[...]
    c = c_ref[0] + c_ref[1]
    cnt = jnp.maximum(c[:, 0:1], 1.0)
    mean_lo = s_ref[0] / cnt
    mean_hi = s_ref[1] / cnt
    w = w_ref[...]
    acc = jnp.dot(x, w[0:_D], preferred_element_type=jnp.float32)
    acc = acc + jnp.dot(mean_lo, w[_D:_D + _DH],
                        preferred_element_type=jnp.float32)
    acc = acc + jnp.dot(mean_hi, w[_D + _DH:2 * _D],
                        preferred_element_type=jnp.float32)
    o_ref[...] = acc + b_ref[...] + res_ref[0, 0]


def _tc_finish(res, nodes, sums, cnts, W, b2d):
    return pl.pallas_call(
        _tc_body,
        grid=(_N // _MB,),
        in_specs=[
            pl.BlockSpec(memory_space=pltpu.SMEM),
            pl.BlockSpec((_MB, _D), lambda i: (i, 0)),
            pl.BlockSpec((_NC, _MB, _DH), lambda i: (0, i, 0)),
            pl.BlockSpec((_NC, _MB, _CW), lambda i: (0, i, 0)),
            pl.BlockSpec((2 * _D, _D), lambda i: (0, 0)),
            pl.BlockSpec((1, _D), lambda i: (0, 0)),
        ],
        out_specs=pl.BlockSpec((_MB, _D), lambda i: (i, 0)),
        out_shape=jax.ShapeDtypeStruct((_N, _D), jnp.float32),
    )(res, nodes, sums, cnts, W, b2d)


def kernel(nodes, senders, receivers, n_node, W, b):
    nodes_halves = jnp.stack([nodes[:, :_DH], nodes[:, _DH:]])
    senders3d = senders.reshape(_NS, _CHUNKS, _B)
    receivers3d = receivers.reshape(_NS, _CHUNKS, _B)
    sums, cnts = _sc_aggregate(nodes_halves, senders3d, receivers3d)
    res = (jnp.asarray(n_node, jnp.float32) - jnp.float32(_N)).reshape(1, 1)
    return _tc_finish(res, nodes, sums, cnts, W, b.reshape(1, _D))
